# Initial kernel scaffold; baseline (speedup 1.0000x reference)
import jax
import jax.numpy as jnp
from jax.experimental import pallas as pl


def kernel(pos_head, pos_relation, pos_tail, neg_head, neg_relation, neg_tail, entity_emb, relation_emb):
    raise NotImplementedError("write your pallas kernel here")


# speedup vs baseline: 1.6424x; 1.6424x over previous
import functools

import jax
import jax.numpy as jnp
from jax import lax
from jax.experimental import pallas as pl
from jax.experimental.pallas import tpu as pltpu
from jax.experimental.pallas import tpu_sc as plsc

_NC = 2
_NS = 16
_L = 16
_NW = _NC * _NS
_BATCH = 16384
_DIM = 64
_BPW = _BATCH // _NW
_CHUNK = 128
_NCHUNK = _BPW // _CHUNK
_NGROUP = _CHUNK // _L
_GAMMA = 1.0


def _vsqrt(x):
    x = jnp.maximum(x, jnp.float32(1e-30))
    i = lax.bitcast_convert_type(x, jnp.int32)
    y = lax.bitcast_convert_type(jnp.int32(0x5F3759DF) - (i >> 1), jnp.float32)
    for _ in range(3):
        y = y * (jnp.float32(1.5) - jnp.float32(0.5) * x * y * y)
    return x * y


def _hsum(v, s):
    for off in (8, 4, 2, 1):
        s[pl.ds(0, _L)] = v
        v = v + s[pl.ds(off, _L)]
    return v[0]


def _body(ph, pr, pt, nh, nr, nt, ent, rel, out,
          iph, ipr, ipt, inh, inr, int_,
          bhp, brp, btp, bhn, brn, btn,
          sp_s, sn_s, accv, sems):
    wid = lax.axis_index("s") * _NC + lax.axis_index("c")
    base = wid * _BPW

    pltpu.sync_copy(ph.at[pl.ds(base, _BPW)], iph)
    pltpu.sync_copy(pr.at[pl.ds(base, _BPW)], ipr)
    pltpu.sync_copy(pt.at[pl.ds(base, _BPW)], ipt)
    pltpu.sync_copy(nh.at[pl.ds(base, _BPW)], inh)
    pltpu.sync_copy(nr.at[pl.ds(base, _BPW)], inr)
    pltpu.sync_copy(nt.at[pl.ds(base, _BPW)], int_)

    zero = jnp.zeros((_L,), jnp.float32)
    sp_s[pl.ds(_L, _L)] = zero
    sn_s[pl.ds(_L, _L)] = zero
    acc0 = zero

    def chunk_body(c, acc):
        sl = pl.ds(c * _CHUNK, _CHUNK)
        cps = [
            pltpu.async_copy(ent.at[iph.at[sl]], bhp, sems.at[0]),
            pltpu.async_copy(rel.at[ipr.at[sl]], brp, sems.at[1]),
            pltpu.async_copy(ent.at[ipt.at[sl]], btp, sems.at[2]),
            pltpu.async_copy(ent.at[inh.at[sl]], bhn, sems.at[3]),
            pltpu.async_copy(rel.at[inr.at[sl]], brn, sems.at[4]),
            pltpu.async_copy(ent.at[int_.at[sl]], btn, sems.at[5]),
        ]
        for cp in cps:
            cp.wait()

        liota = lax.broadcasted_iota(jnp.int32, (_L,), 0)

        def group_body(g, acc):
            vsp = jnp.zeros((_L,), jnp.float32)
            vsn = jnp.zeros((_L,), jnp.float32)
            for i in range(_L):
                r = g * _L + i
                accp = jnp.zeros((_L,), jnp.float32)
                accn = jnp.zeros((_L,), jnp.float32)
                for j in range(_DIM // _L):
                    dsl = pl.ds(j * _L, _L)
                    dp = bhp[r, dsl] + brp[r, dsl] - btp[r, dsl]
                    dn = bhn[r, dsl] + brn[r, dsl] - btn[r, dsl]
                    accp = accp + dp * dp
                    accn = accn + dn * dn
                lane = liota == jnp.int32(i)
                vsp = jnp.where(lane, jnp.full((_L,), _hsum(accp, sp_s)), vsp)
                vsn = jnp.where(lane, jnp.full((_L,), _hsum(accn, sn_s)), vsn)
            margin = jnp.float32(_GAMMA) + _vsqrt(vsp) - _vsqrt(vsn)
            return acc + jnp.maximum(margin, jnp.float32(0.0))

        return lax.fori_loop(0, _NGROUP, group_body, acc)

    acc = lax.fori_loop(0, _NCHUNK, chunk_body, acc0)
    accv[...] = acc
    pltpu.sync_copy(accv, out.at[wid])


@functools.partial(
    pl.kernel,
    out_type=jax.ShapeDtypeStruct((_NW, _L), jnp.float32),
    mesh=plsc.VectorSubcoreMesh(
        core_axis_name="c", subcore_axis_name="s",
        num_cores=_NC, num_subcores=_NS),
    compiler_params=pltpu.CompilerParams(use_tc_tiling_on_sc=False),
    scratch_types=[
        pltpu.VMEM((_BPW,), jnp.int32),
        pltpu.VMEM((_BPW,), jnp.int32),
        pltpu.VMEM((_BPW,), jnp.int32),
        pltpu.VMEM((_BPW,), jnp.int32),
        pltpu.VMEM((_BPW,), jnp.int32),
        pltpu.VMEM((_BPW,), jnp.int32),
        pltpu.VMEM((_CHUNK, _DIM), jnp.float32),
        pltpu.VMEM((_CHUNK, _DIM), jnp.float32),
        pltpu.VMEM((_CHUNK, _DIM), jnp.float32),
        pltpu.VMEM((_CHUNK, _DIM), jnp.float32),
        pltpu.VMEM((_CHUNK, _DIM), jnp.float32),
        pltpu.VMEM((_CHUNK, _DIM), jnp.float32),
        pltpu.VMEM((2 * _L,), jnp.float32),
        pltpu.VMEM((2 * _L,), jnp.float32),
        pltpu.VMEM((_L,), jnp.float32),
        pltpu.SemaphoreType.DMA((6,)),
    ],
)
def _transe_sc(*refs):
    _body(*refs[:9], *refs[9:15], *refs[15:21], refs[21], refs[22],
          refs[23], refs[24])


def kernel(pos_head, pos_relation, pos_tail,
           neg_head, neg_relation, neg_tail,
           entity_emb, relation_emb):
    partials = _transe_sc(pos_head, pos_relation, pos_tail,
                          neg_head, neg_relation, neg_tail,
                          entity_emb, relation_emb)
    return jnp.sum(partials)
